# baseline (speedup 1.0000x reference)
import jax
import jax.numpy as jnp
import numpy as np
from jax.experimental import pallas as pl
from jax.experimental.pallas import tpu as pltpu

_H = 1024
_NUM_CELLS = 6
_NUM_LAYERS = 12
_TBL = 16
_NPAD = 16


def _threefry2x32(k0, k1, x0, x1):
    rot = ((13, 15, 26, 6), (17, 29, 16, 24))

    def rotl(x, d):
        return ((x << np.uint32(d)) | (x >> np.uint32(32 - d))).astype(
            np.uint32)

    ks = (k0, k1, (k0 ^ k1 ^ np.uint32(0x1BD11BDA)).astype(np.uint32))
    x0 = (x0 + ks[0]).astype(np.uint32)
    x1 = (x1 + ks[1]).astype(np.uint32)
    for i in range(5):
        for d in rot[i % 2]:
            x0 = (x0 + x1).astype(np.uint32)
            x1 = rotl(x1, d) ^ x0
        x0 = (x0 + ks[(i + 1) % 3]).astype(np.uint32)
        x1 = (x1 + ks[(i + 2) % 3] + np.uint32(i + 1)).astype(np.uint32)
    return x0, x1


def _random_bits(k0, k1, n):
    o0, o1 = _threefry2x32(k0, k1, np.zeros(n, np.uint32),
                           np.arange(n, dtype=np.uint32))
    return o0 ^ o1


def _uniform_draws(k0, k1, n):
    bits = _random_bits(k0, k1, n)
    fb = ((bits >> np.uint32(9)) | np.uint32(0x3F800000)).view(np.float32)
    tiny = np.float32(np.finfo(np.float32).tiny)
    f = (fb - np.float32(1.0)).astype(np.float32)
    return np.maximum(tiny, (f * (np.float32(1.0) - tiny) + tiny).astype(
        np.float32))


def _noise_table():
    key0, key1 = np.uint32(0), np.uint32(42)
    tbl = np.zeros((40, _NPAD), np.float32)
    ctr = 0
    row = 0
    for layer_id in range(2, _NUM_LAYERS):
        for n in (layer_id, layer_id, _NUM_CELLS, _NUM_CELLS):
            ctr += 1
            f0, f1 = _threefry2x32(key0, key1,
                                   np.zeros(1, np.uint32),
                                   np.full(1, ctr, np.uint32))
            tbl[row, :n] = _uniform_draws(f0[0], f1[0], n)
            row += 1
    return tbl.reshape(40, _NPAD, 1)


_GUM = _noise_table()
_GUM_ROW = _GUM.reshape(40, 1, _NPAD)


def _mm(x, wt):
    return jnp.dot(x, wt, preferred_element_type=jnp.float32)


def _ctrl_kernel(wih_ref, whh_ref, attn1_ref, attn2_ref, attnv_ref, wlin_ref,
                 blin_ref, emb_ref, b2_ref, gum_ref, gumrow_ref, out_ref,
                 allwh_ref, allhw_ref):
    wih = wih_ref[...]
    whh = whh_ref[...]
    attn1 = attn1_ref[...]
    attn2 = attn2_ref[...]
    attnv = attnv_ref[...]
    wlin = wlin_ref[...]
    blin = blin_ref[...]
    b2 = b2_ref[...]

    allwh_ref[...] = jnp.zeros((_TBL, _H), jnp.float32)
    allhw_ref[...] = jnp.zeros((_TBL, 4 * _H), jnp.float32)

    iota_cat = jax.lax.broadcasted_iota(jnp.int32, (_NPAD, 1), 0)
    iota_row = jax.lax.broadcasted_iota(jnp.int32, (1, _NPAD), 1)
    iota_tbl = jax.lax.broadcasted_iota(jnp.int32, (1, _TBL), 1)
    iota_out = jax.lax.broadcasted_iota(jnp.int32, (1, 128), 1)

    def lstm(gx, h, c):
        gates = gx + _mm(h.astype(jnp.bfloat16), whh)
        i = jax.nn.sigmoid(gates[:, 0 * _H:1 * _H])
        f = jax.nn.sigmoid(gates[:, 1 * _H:2 * _H])
        g = jnp.tanh(gates[:, 2 * _H:3 * _H])
        o = jax.nn.sigmoid(gates[:, 3 * _H:4 * _H])
        c2 = f * c + i * g
        h2 = o * jnp.tanh(c2)
        return h2, c2

    def log_softmax_singleton(col):
        m = jnp.max(col, axis=1, keepdims=True)
        e = jnp.exp(col - m)
        probs = e / jnp.sum(e, axis=1, keepdims=True)
        return jnp.log(probs)

    def sample(logits_col, step):
        vals = logits_col + gum_ref[step]
        m = jnp.max(vals)
        return jnp.min(jnp.where(vals == m, iota_cat, _NPAD))

    def sample_row(logits_row, step):
        vals = logits_row + gumrow_ref[step]
        m = jnp.max(vals)
        return jnp.min(jnp.where(vals == m, iota_row, _NPAD))

    h = jnp.zeros((1, _H), jnp.float32)
    c = jnp.zeros((1, _H), jnp.float32)
    gx = _mm(emb_ref[...].astype(jnp.bfloat16), wih) + b2
    seq = jnp.zeros((1, 128), jnp.int32)

    rows = 0
    pending = []
    for _ in range(2):
        h, c = lstm(gx, h, c)
        hb = h.astype(jnp.bfloat16)
        pending.append((rows, _mm(hb, attn1), _mm(hb, wih) + b2))
        rows += 1

    step = 0
    for layer_id in range(2, _NUM_LAYERS):
        for r, wh_row, hw_row in pending:
            allwh_ref[pl.ds(r, 1), :] = wh_row
            allhw_ref[pl.ds(r, 1), :] = hw_row
        pending = []
        for _ in range(2):
            h, c = lstm(gx, h, c)
            hb = h.astype(jnp.bfloat16)
            q = allwh_ref[...] + _mm(hb, attn2)
            align = jnp.sum(jnp.tanh(q) * attnv, axis=1, keepdims=True)
            logp = log_softmax_singleton(align)
            logits = jnp.where(iota_cat < layer_id, logp, -1e30)
            skip_idx = sample(logits, step)
            seq = jnp.where(iota_out == step, skip_idx, seq)
            step += 1
            gx = allhw_ref[pl.ds(skip_idx, 1), :]
            if rows < 11:
                pending.append((rows, _mm(hb, attn1), _mm(hb, wih) + b2))
            rows += 1
        for _ in range(2):
            h, c = lstm(gx, h, c)
            logit = _mm(h.astype(jnp.bfloat16), wlin) + blin
            m = jnp.max(logit, axis=0, keepdims=True)
            e = jnp.exp(logit - m)
            logp = jnp.log(e / jnp.sum(e, axis=0, keepdims=True))
            logits = jnp.where(iota_row < _NUM_CELLS, logp, -1e30)
            op_idx = sample_row(logits, step)
            seq = jnp.where(iota_out == step, op_idx, seq)
            step += 1

    out_ref[...] = seq


@jax.jit
def kernel(W_ih, W_hh, b_ih, b_hh, W_lin, b_lin, emb, attn1, attn2, attnv):
    wih_bf = W_ih.T.astype(jnp.bfloat16)
    whh_bf = W_hh.T.astype(jnp.bfloat16)
    b2 = (b_ih + b_hh).reshape(1, 4 * _H)
    wlin_p = jnp.zeros((_H, _NPAD), jnp.bfloat16).at[:, :_NUM_CELLS].set(
        W_lin.T.astype(jnp.bfloat16))
    blin_p = jnp.zeros((1, _NPAD), jnp.float32).at[0, :_NUM_CELLS].set(b_lin)

    out = pl.pallas_call(
        _ctrl_kernel,
        out_shape=jax.ShapeDtypeStruct((1, 128), jnp.int32),
        scratch_shapes=[
            pltpu.VMEM((_TBL, _H), jnp.float32),
            pltpu.VMEM((_TBL, 4 * _H), jnp.float32),
        ],
        compiler_params=pltpu.CompilerParams(
            vmem_limit_bytes=100 * 1024 * 1024),
    )(wih_bf, whh_bf, attn1.T.astype(jnp.bfloat16),
      attn2.T.astype(jnp.bfloat16), attnv, wlin_p, blin_p, emb, b2,
      jnp.asarray(_GUM), jnp.asarray(_GUM_ROW))
    return out[0, :40]


# speedup vs baseline: 1.0324x; 1.0324x over previous
import jax
import jax.numpy as jnp
import numpy as np
from jax.experimental import pallas as pl
from jax.experimental.pallas import tpu as pltpu

_H = 1024
_NUM_CELLS = 6
_NUM_LAYERS = 12
_TBL = 16
_NPAD = 16


def _threefry2x32(k0, k1, x0, x1):
    rot = ((13, 15, 26, 6), (17, 29, 16, 24))

    def rotl(x, d):
        return ((x << np.uint32(d)) | (x >> np.uint32(32 - d))).astype(
            np.uint32)

    ks = (k0, k1, (k0 ^ k1 ^ np.uint32(0x1BD11BDA)).astype(np.uint32))
    x0 = (x0 + ks[0]).astype(np.uint32)
    x1 = (x1 + ks[1]).astype(np.uint32)
    for i in range(5):
        for d in rot[i % 2]:
            x0 = (x0 + x1).astype(np.uint32)
            x1 = rotl(x1, d) ^ x0
        x0 = (x0 + ks[(i + 1) % 3]).astype(np.uint32)
        x1 = (x1 + ks[(i + 2) % 3] + np.uint32(i + 1)).astype(np.uint32)
    return x0, x1


def _random_bits(k0, k1, n):
    o0, o1 = _threefry2x32(k0, k1, np.zeros(n, np.uint32),
                           np.arange(n, dtype=np.uint32))
    return o0 ^ o1


def _uniform_draws(k0, k1, n):
    bits = _random_bits(k0, k1, n)
    fb = ((bits >> np.uint32(9)) | np.uint32(0x3F800000)).view(np.float32)
    tiny = np.float32(np.finfo(np.float32).tiny)
    f = (fb - np.float32(1.0)).astype(np.float32)
    return np.maximum(tiny, (f * (np.float32(1.0) - tiny) + tiny).astype(
        np.float32))


def _noise_table():
    key0, key1 = np.uint32(0), np.uint32(42)
    tbl = np.zeros((40, _NPAD), np.float32)
    ctr = 0
    row = 0
    for layer_id in range(2, _NUM_LAYERS):
        for n in (layer_id, layer_id, _NUM_CELLS, _NUM_CELLS):
            ctr += 1
            f0, f1 = _threefry2x32(key0, key1,
                                   np.zeros(1, np.uint32),
                                   np.full(1, ctr, np.uint32))
            tbl[row, :n] = _uniform_draws(f0[0], f1[0], n)
            row += 1
    return tbl.reshape(40, _NPAD, 1)


_GUM = _noise_table()


def _mm(x, wt):
    return jnp.dot(x, wt, preferred_element_type=jnp.float32)


def _ctrl_kernel(wih_ref, whh_ref, attn1_ref, attn2_ref, attnv_ref, wlin_ref,
                 blin_ref, emb_ref, b2_ref, gum_ref, out_ref,
                 allwh_ref, allhw_ref):
    wih = wih_ref[...]
    whh = whh_ref[...]
    attn1 = attn1_ref[...]
    attn2 = attn2_ref[...]
    attnv = attnv_ref[...]
    wlin = wlin_ref[...]
    blin = blin_ref[...]
    b2 = b2_ref[...]

    allwh_ref[...] = jnp.zeros((_TBL, _H), jnp.float32)
    allhw_ref[...] = jnp.zeros((_TBL, 4 * _H), jnp.float32)

    iota_cat = jax.lax.broadcasted_iota(jnp.int32, (_NPAD, 1), 0)
    iota_tbl = jax.lax.broadcasted_iota(jnp.int32, (1, _TBL), 1)
    iota_out = jax.lax.broadcasted_iota(jnp.int32, (1, 128), 1)

    def sigm(x):
        return 0.5 * jnp.tanh(0.5 * x) + 0.5

    def lstm(gx, h, c):
        gates = gx + _mm(h.astype(jnp.bfloat16), whh)
        i = sigm(gates[:, 0 * _H:1 * _H])
        f = sigm(gates[:, 1 * _H:2 * _H])
        g = jnp.tanh(gates[:, 2 * _H:3 * _H])
        o = sigm(gates[:, 3 * _H:4 * _H])
        c2 = f * c + i * g
        h2 = o * jnp.tanh(c2)
        return h2, c2

    def log_softmax_singleton(col):
        m = jnp.max(col, axis=1, keepdims=True)
        e = jnp.exp(col - m)
        probs = e / jnp.sum(e, axis=1, keepdims=True)
        return jnp.log(probs)

    def sample(logits_col, step):
        vals = logits_col + gum_ref[step]
        m = jnp.max(vals)
        return jnp.min(jnp.where(vals == m, iota_cat, _NPAD))

    h = jnp.zeros((1, _H), jnp.float32)
    c = jnp.zeros((1, _H), jnp.float32)
    gx = _mm(emb_ref[...].astype(jnp.bfloat16), wih) + b2
    seq = jnp.zeros((1, 128), jnp.int32)

    rows = 0
    pending = []
    for _ in range(2):
        h, c = lstm(gx, h, c)
        hb = h.astype(jnp.bfloat16)
        pending.append((rows, _mm(hb, attn1), _mm(hb, wih) + b2))
        rows += 1

    step = 0
    for layer_id in range(2, _NUM_LAYERS):
        for r, wh_row, hw_row in pending:
            allwh_ref[pl.ds(r, 1), :] = wh_row
            allhw_ref[pl.ds(r, 1), :] = hw_row
        pending = []
        for _ in range(2):
            h, c = lstm(gx, h, c)
            hb = h.astype(jnp.bfloat16)
            q = allwh_ref[...] + _mm(hb, attn2)
            align = jnp.sum(jnp.tanh(q) * attnv, axis=1, keepdims=True)
            logp = log_softmax_singleton(align)
            logits = jnp.where(iota_cat < layer_id, logp, -1e30)
            skip_idx = sample(logits, step)
            seq = jnp.where(iota_out == step, skip_idx, seq)
            step += 1
            gx = allhw_ref[pl.ds(skip_idx, 1), :]
            if rows < 11:
                pending.append((rows, _mm(hb, attn1), _mm(hb, wih) + b2))
            rows += 1
        for _ in range(2):
            h, c = lstm(gx, h, c)
            logit = jnp.sum(wlin * h, axis=1, keepdims=True) + blin
            logp = log_softmax_singleton(logit)
            logits = jnp.where(iota_cat < _NUM_CELLS, logp, -1e30)
            op_idx = sample(logits, step)
            seq = jnp.where(iota_out == step, op_idx, seq)
            step += 1

    out_ref[...] = seq


@jax.jit
def kernel(W_ih, W_hh, b_ih, b_hh, W_lin, b_lin, emb, attn1, attn2, attnv):
    wih_bf = W_ih.T.astype(jnp.bfloat16)
    whh_bf = W_hh.T.astype(jnp.bfloat16)
    b2 = (b_ih + b_hh).reshape(1, 4 * _H)
    wlin_p = jnp.zeros((_NPAD, _H), jnp.float32).at[:_NUM_CELLS].set(W_lin)
    blin_p = jnp.zeros((_NPAD, 1), jnp.float32).at[:_NUM_CELLS, 0].set(b_lin)

    out = pl.pallas_call(
        _ctrl_kernel,
        out_shape=jax.ShapeDtypeStruct((1, 128), jnp.int32),
        scratch_shapes=[
            pltpu.VMEM((_TBL, _H), jnp.float32),
            pltpu.VMEM((_TBL, 4 * _H), jnp.float32),
        ],
        compiler_params=pltpu.CompilerParams(
            vmem_limit_bytes=100 * 1024 * 1024),
    )(wih_bf, whh_bf, attn1.T.astype(jnp.bfloat16),
      attn2.T.astype(jnp.bfloat16), attnv, wlin_p, blin_p, emb, b2,
      jnp.asarray(_GUM))
    return out[0, :40]


# speedup vs baseline: 1.0429x; 1.0102x over previous
import jax
import jax.numpy as jnp
import numpy as np
from jax.experimental import pallas as pl
from jax.experimental.pallas import tpu as pltpu

_H = 1024
_NUM_CELLS = 6
_NUM_LAYERS = 12
_TBL = 16
_NPAD = 16


def _threefry2x32(k0, k1, x0, x1):
    rot = ((13, 15, 26, 6), (17, 29, 16, 24))

    def rotl(x, d):
        return ((x << np.uint32(d)) | (x >> np.uint32(32 - d))).astype(
            np.uint32)

    ks = (k0, k1, (k0 ^ k1 ^ np.uint32(0x1BD11BDA)).astype(np.uint32))
    x0 = (x0 + ks[0]).astype(np.uint32)
    x1 = (x1 + ks[1]).astype(np.uint32)
    for i in range(5):
        for d in rot[i % 2]:
            x0 = (x0 + x1).astype(np.uint32)
            x1 = rotl(x1, d) ^ x0
        x0 = (x0 + ks[(i + 1) % 3]).astype(np.uint32)
        x1 = (x1 + ks[(i + 2) % 3] + np.uint32(i + 1)).astype(np.uint32)
    return x0, x1


def _random_bits(k0, k1, n):
    o0, o1 = _threefry2x32(k0, k1, np.zeros(n, np.uint32),
                           np.arange(n, dtype=np.uint32))
    return o0 ^ o1


def _uniform_draws(k0, k1, n):
    bits = _random_bits(k0, k1, n)
    fb = ((bits >> np.uint32(9)) | np.uint32(0x3F800000)).view(np.float32)
    tiny = np.float32(np.finfo(np.float32).tiny)
    f = (fb - np.float32(1.0)).astype(np.float32)
    return np.maximum(tiny, (f * (np.float32(1.0) - tiny) + tiny).astype(
        np.float32))


def _noise_table():
    key0, key1 = np.uint32(0), np.uint32(42)
    tbl = np.zeros((40, _NPAD), np.float32)
    ctr = 0
    row = 0
    for layer_id in range(2, _NUM_LAYERS):
        for n in (layer_id, layer_id, _NUM_CELLS, _NUM_CELLS):
            ctr += 1
            f0, f1 = _threefry2x32(key0, key1,
                                   np.zeros(1, np.uint32),
                                   np.full(1, ctr, np.uint32))
            tbl[row, :n] = _uniform_draws(f0[0], f1[0], n)
            row += 1
    return tbl.reshape(40, _NPAD, 1)


_GUM = _noise_table()


def _mm(x, wt):
    return jnp.dot(x, wt, preferred_element_type=jnp.float32)


def _ctrl_kernel(wih_ref, whh_ref, attn1_ref, attn2_ref, attnv_ref, wlin_ref,
                 blin_ref, emb_ref, b2_ref, gum_ref, out_ref,
                 allwh_ref, allhw_ref):
    wih = wih_ref[...]
    whh = whh_ref[...]
    attn1 = attn1_ref[...]
    attn2 = attn2_ref[...]
    attnv = attnv_ref[...]
    wlin = wlin_ref[...]
    blin = blin_ref[...]
    b2 = b2_ref[...]

    allwh_ref[...] = jnp.zeros((_TBL, _H), jnp.float32)
    allhw_ref[...] = jnp.zeros((_TBL, 4 * _H), jnp.float32)

    iota_cat = jax.lax.broadcasted_iota(jnp.int32, (_NPAD, 1), 0)
    iota_tbl = jax.lax.broadcasted_iota(jnp.int32, (1, _TBL), 1)
    iota_out = jax.lax.broadcasted_iota(jnp.int32, (1, 128), 1)

    def sigm(x):
        return 0.5 * jnp.tanh(0.5 * x) + 0.5

    def lstm(gx, h, c):
        gates = gx + _mm(h.astype(jnp.float8_e4m3fn), whh)
        i = sigm(gates[:, 0 * _H:1 * _H])
        f = sigm(gates[:, 1 * _H:2 * _H])
        g = jnp.tanh(gates[:, 2 * _H:3 * _H])
        o = sigm(gates[:, 3 * _H:4 * _H])
        c2 = f * c + i * g
        h2 = o * jnp.tanh(c2)
        return h2, c2

    def log_softmax_singleton(col):
        m = jnp.max(col, axis=1, keepdims=True)
        e = jnp.exp(col - m)
        probs = e / jnp.sum(e, axis=1, keepdims=True)
        return jnp.log(probs)

    def sample(logits_col, step):
        vals = logits_col + gum_ref[step]
        m = jnp.max(vals)
        return jnp.min(jnp.where(vals == m, iota_cat, _NPAD))

    h = jnp.zeros((1, _H), jnp.float32)
    c = jnp.zeros((1, _H), jnp.float32)
    gx = _mm(emb_ref[...].astype(jnp.float8_e4m3fn), wih) + b2
    seq = jnp.zeros((1, 128), jnp.int32)

    rows = 0
    pending = []
    for _ in range(2):
        h, c = lstm(gx, h, c)
        hb = h.astype(jnp.bfloat16)
        pending.append((rows, _mm(hb, attn1),
                _mm(hb.astype(jnp.float8_e4m3fn), wih) + b2))
        rows += 1

    step = 0
    for layer_id in range(2, _NUM_LAYERS):
        for r, wh_row, hw_row in pending:
            allwh_ref[pl.ds(r, 1), :] = wh_row
            allhw_ref[pl.ds(r, 1), :] = hw_row
        pending = []
        for _ in range(2):
            h, c = lstm(gx, h, c)
            hb = h.astype(jnp.bfloat16)
            q = allwh_ref[...] + _mm(hb, attn2)
            align = jnp.sum(jnp.tanh(q) * attnv, axis=1, keepdims=True)
            logp = log_softmax_singleton(align)
            logits = jnp.where(iota_cat < layer_id, logp, -1e30)
            skip_idx = sample(logits, step)
            seq = jnp.where(iota_out == step, skip_idx, seq)
            step += 1
            gx = allhw_ref[pl.ds(skip_idx, 1), :]
            if rows < 11:
                pending.append((rows, _mm(hb, attn1),
                _mm(hb.astype(jnp.float8_e4m3fn), wih) + b2))
            rows += 1
        for _ in range(2):
            h, c = lstm(gx, h, c)
            logit = jnp.sum(wlin * h, axis=1, keepdims=True) + blin
            logp = log_softmax_singleton(logit)
            logits = jnp.where(iota_cat < _NUM_CELLS, logp, -1e30)
            op_idx = sample(logits, step)
            seq = jnp.where(iota_out == step, op_idx, seq)
            step += 1

    out_ref[...] = seq


@jax.jit
def kernel(W_ih, W_hh, b_ih, b_hh, W_lin, b_lin, emb, attn1, attn2, attnv):
    wih_bf = W_ih.T.astype(jnp.float8_e4m3fn)
    whh_bf = W_hh.T.astype(jnp.float8_e4m3fn)
    b2 = (b_ih + b_hh).reshape(1, 4 * _H)
    wlin_p = jnp.zeros((_NPAD, _H), jnp.float32).at[:_NUM_CELLS].set(W_lin)
    blin_p = jnp.zeros((_NPAD, 1), jnp.float32).at[:_NUM_CELLS, 0].set(b_lin)

    out = pl.pallas_call(
        _ctrl_kernel,
        out_shape=jax.ShapeDtypeStruct((1, 128), jnp.int32),
        scratch_shapes=[
            pltpu.VMEM((_TBL, _H), jnp.float32),
            pltpu.VMEM((_TBL, 4 * _H), jnp.float32),
        ],
        compiler_params=pltpu.CompilerParams(
            vmem_limit_bytes=100 * 1024 * 1024),
    )(wih_bf, whh_bf, attn1.T.astype(jnp.bfloat16),
      attn2.T.astype(jnp.bfloat16), attnv, wlin_p, blin_p, emb, b2,
      jnp.asarray(_GUM))
    return out[0, :40]
